# Initial kernel scaffold; baseline (speedup 1.0000x reference)
import jax
import jax.numpy as jnp
from jax.experimental import pallas as pl


def kernel(node_features, edge_features, global_features, node_graph_ids, edge_graph_ids, W_node, W_edges, W_global, bias):
    raise NotImplementedError("write your pallas kernel here")


# speedup vs baseline: 6.1473x; 6.1473x over previous
import jax
import jax.numpy as jnp
from jax.experimental import pallas as pl
from jax.experimental.pallas import tpu as pltpu

NUM_GRAPHS = 128
N_NODES = 100000
N_EDGES = 1600000
D_NODE = 128
D_EDGE = 16
D_GLOBAL = 64
D_OUT = 128

BN = 1000
BE = 8000


def _segsum_body(feat_ref, ids_ref, out_ref):
    step = pl.program_id(0)

    @pl.when(step == 0)
    def _():
        out_ref[...] = jnp.zeros_like(out_ref)

    ids = ids_ref[0, 0, :]
    rows = feat_ref[...]
    seg = jax.lax.broadcasted_iota(jnp.int32, (NUM_GRAPHS, ids.shape[0]), 0)
    onehot = jnp.where(seg == ids[None, :], 1.0, 0.0).astype(jnp.float32)
    out_ref[...] += jax.lax.dot_general(
        onehot, rows, (((1,), (0,)), ((), ())),
        preferred_element_type=jnp.float32)


def _segment_sum(feat, ids, block_rows):
    n, d = feat.shape
    grid = n // block_rows
    ids3 = ids.reshape(grid, 1, block_rows)
    return pl.pallas_call(
        _segsum_body,
        grid=(grid,),
        in_specs=[
            pl.BlockSpec((block_rows, d), lambda i: (i, 0)),
            pl.BlockSpec((1, 1, block_rows), lambda i: (i, 0, 0)),
        ],
        out_specs=pl.BlockSpec((NUM_GRAPHS, d), lambda i: (0, 0)),
        out_shape=jax.ShapeDtypeStruct((NUM_GRAPHS, d), jnp.float32),
    )(feat, ids3)


def _final_body(an_ref, ae_ref, g_ref, wn_ref, we_ref, wg_ref, b_ref, out_ref):
    acc = jax.lax.dot_general(
        an_ref[...], wn_ref[...], (((1,), (1,)), ((), ())),
        preferred_element_type=jnp.float32)
    acc += jax.lax.dot_general(
        ae_ref[...], we_ref[...], (((1,), (1,)), ((), ())),
        preferred_element_type=jnp.float32)
    acc += jax.lax.dot_general(
        g_ref[...], wg_ref[...], (((1,), (1,)), ((), ())),
        preferred_element_type=jnp.float32)
    out_ref[...] = acc + b_ref[...]


def kernel(node_features, edge_features, global_features, node_graph_ids,
           edge_graph_ids, W_node, W_edges, W_global, bias):
    agg_nodes = _segment_sum(node_features, node_graph_ids, BN)
    agg_edges = _segment_sum(edge_features, edge_graph_ids, BE)
    return pl.pallas_call(
        _final_body,
        out_shape=jax.ShapeDtypeStruct((NUM_GRAPHS, D_OUT), jnp.float32),
    )(agg_nodes, agg_edges, global_features, W_node, W_edges, W_global,
      bias.reshape(1, D_OUT))
